# baseline (speedup 1.0000x reference)
import functools

import jax
import jax.numpy as jnp
from jax import lax
from jax.experimental import pallas as pl
from jax.experimental.pallas import tpu as pltpu
from jax.experimental.pallas import tpu_sc as plsc

D = 64
B = 16384
NC = 2
NS = 16
NW = NC * NS
BPW = B // NW
CHUNK = 128
NCH = BPW // CHUNK
L = 16


_mesh = plsc.VectorSubcoreMesh(core_axis_name="c", subcore_axis_name="s")


_SC_SCRATCH = [
    pltpu.VMEM((BPW,), jnp.int32),
    pltpu.VMEM((BPW, D), jnp.float32),
    pltpu.VMEM((8,), jnp.int32),
    pltpu.VMEM((1, D), jnp.float32),
    pltpu.VMEM((BPW,), jnp.float32),
    pltpu.SemaphoreType.DMA,
]


def _sc_dots_body(iw_hbm, ctx_hbm, win_hbm, wctx_hbm, out_hbm,
             idx_v, rows_v, iwidx_v, inp_v, dots_v, sem):
    wid = lax.axis_index("s") * NC + lax.axis_index("c")
    base = wid * BPW

    pltpu.sync_copy(ctx_hbm.at[pl.ds(base, BPW)], idx_v)
    pltpu.sync_copy(iw_hbm, iwidx_v.at[pl.ds(0, 1)])
    pltpu.async_copy(win_hbm.at[iwidx_v.at[pl.ds(0, 1)]], inp_v, sem).wait()

    cps = [
        pltpu.async_copy(
            wctx_hbm.at[idx_v.at[pl.ds(j * CHUNK, CHUNK)]],
            rows_v.at[pl.ds(j * CHUNK, CHUNK)],
            sem,
        )
        for j in range(NCH)
    ]
    for cp in cps:
        cp.wait()

    def _round_bf16(x):
        u = plsc.bitcast(x, jnp.int32)
        r = u + jnp.int32(0x7FFF) + ((u >> 16) & 1)
        return plsc.bitcast(r & jnp.int32(-65536), jnp.float32)

    inp_vecs = [
        _round_bf16(inp_v[0, pl.ds(c * L, L)]) for c in range(D // L)
    ]

    def group(g, carry):
        rid = lax.iota(jnp.int32, L) + g * L
        acc = jnp.zeros((L,), jnp.float32)
        for d in range(D):
            col = plsc.load_gather(rows_v, [rid, jnp.full((L,), d, jnp.int32)])
            acc = acc + _round_bf16(col) * inp_vecs[d // L][d % L]
        dots_v[pl.ds(g * L, L)] = acc
        return carry

    lax.fori_loop(0, BPW // L, group, 0)

    pltpu.sync_copy(dots_v, out_hbm.at[pl.ds(base, BPW)])


_sc_dots = pl.kernel(
    _sc_dots_body,
    mesh=_mesh,
    out_type=jax.ShapeDtypeStruct((B,), jnp.float32),
    compiler_params=pltpu.CompilerParams(
        needs_layout_passes=False, use_tc_tiling_on_sc=False
    ),
    scratch_types=_SC_SCRATCH,
)


def _softmax_body(x_ref, o_ref):
    x = x_ref[...]
    m = jnp.max(x)
    e = jnp.exp(x - m)
    o_ref[...] = e / jnp.sum(e)


_tc_softmax = pl.pallas_call(
    _softmax_body,
    out_shape=jax.ShapeDtypeStruct((8, B // 8), jnp.float32),
)


def kernel(input_word, context, W_in, W_ctx):
    dots = _sc_dots(input_word, context, W_in, W_ctx)
    scores = _tc_softmax(dots.reshape(8, B // 8))
    return scores.reshape(1, B)


# speedup vs baseline: 4.9864x; 4.9864x over previous
import jax
import jax.numpy as jnp
from jax import lax
from jax.experimental import pallas as pl
from jax.experimental.pallas import tpu as pltpu
from jax.experimental.pallas import tpu_sc as plsc

VOCAB = 1000000
D = 64
B = 16384
NC = 2
NS = 16
NW = NC * NS
BPW = B // NW
L = 16

CW = 4096
GRID = -(-VOCAB // CW)
ROWS = GRID * (CW // 128)


def _tc_scan_body(iw_ref, win_blk, wctx_blk, out_ref):
    lane = iw_ref[0] & 127
    m = lax.broadcasted_iota(jnp.int32, (D, 128), 1) == lane
    inp_col = jnp.sum(jnp.where(m, win_blk[...], 0.0), axis=1, keepdims=True)
    prod = lax.dot_general(inp_col, wctx_blk[...], (((0,), (0,)), ((), ())))
    out_ref[...] = prod.reshape(CW // 128, 128)


_tc_scan = pl.pallas_call(
    _tc_scan_body,
    grid_spec=pltpu.PrefetchScalarGridSpec(
        num_scalar_prefetch=1,
        grid=(GRID,),
        in_specs=[
            pl.BlockSpec((D, 128), lambda g, iw: (0, iw[0] // 128)),
            pl.BlockSpec((D, CW), lambda g, iw: (0, g)),
        ],
        out_specs=pl.BlockSpec((CW // 128, 128), lambda g, iw: (g, 0)),
    ),
    out_shape=jax.ShapeDtypeStruct((ROWS, 128), jnp.float32),
)


_mesh = plsc.VectorSubcoreMesh(core_axis_name="c", subcore_axis_name="s")

_SC_SCRATCH = [
    pltpu.VMEM((BPW,), jnp.int32),
    pltpu.VMEM((BPW, 128), jnp.float32),
    pltpu.VMEM((BPW,), jnp.float32),
    pltpu.SemaphoreType.DMA,
]


def _sc_gather_body(ctx_hbm, dots2_hbm, out_hbm, idx_v, rows_v, dots_v, sem):
    wid = lax.axis_index("s") * NC + lax.axis_index("c")
    base = wid * BPW
    pltpu.sync_copy(ctx_hbm.at[pl.ds(base, BPW)], idx_v)
    riota = lax.iota(jnp.int32, L)

    def fire(g, carry):
        rows16 = idx_v[pl.ds(g * L, L)] >> 7
        pltpu.async_copy(
            dots2_hbm.at[rows16], rows_v.at[pl.ds(g * L, L)], sem
        )
        return carry

    lax.fori_loop(0, BPW // L, fire, 0)

    def drain_extract(g, carry):
        pltpu.make_async_copy(
            dots2_hbm.at[pl.ds(0, L)], rows_v.at[pl.ds(g * L, L)], sem
        ).wait()
        lanes = idx_v[pl.ds(g * L, L)] & 127
        vals = plsc.load_gather(rows_v, [riota + g * L, lanes])
        dots_v[pl.ds(g * L, L)] = vals
        return carry

    lax.fori_loop(0, BPW // L, drain_extract, 0)

    pltpu.sync_copy(dots_v, out_hbm.at[pl.ds(base, BPW)])


_sc_gather = pl.kernel(
    _sc_gather_body,
    mesh=_mesh,
    out_type=jax.ShapeDtypeStruct((B,), jnp.float32),
    compiler_params=pltpu.CompilerParams(needs_layout_passes=False),
    scratch_types=_SC_SCRATCH,
)


def _softmax_body(x_ref, o_ref):
    x = x_ref[...]
    m = jnp.max(x)
    e = jnp.exp(x - m)
    o_ref[...] = e / jnp.sum(e)


_tc_softmax = pl.pallas_call(
    _softmax_body,
    out_shape=jax.ShapeDtypeStruct((8, B // 8), jnp.float32),
)


def kernel(input_word, context, W_in, W_ctx):
    dots2 = _tc_scan(input_word, W_in.T, W_ctx.T)
    dots = _sc_gather(context, dots2)
    scores = _tc_softmax(dots.reshape(8, B // 8))
    return scores.reshape(1, B)


# speedup vs baseline: 7.4029x; 1.4846x over previous
import jax
import jax.numpy as jnp
from jax import lax
from jax.experimental import pallas as pl
from jax.experimental.pallas import tpu as pltpu
from jax.experimental.pallas import tpu_sc as plsc

VOCAB = 1000000
D = 64
B = 16384
NC = 2
NS = 16
NW = NC * NS
BPW = B // NW
L = 16

CW = 8192
GRID = -(-VOCAB // CW)
ROWS = GRID * (CW // 128)


def _tc_scan_body(iw_ref, win_blk, wctx_blk, out_ref, inp_ref):
    @pl.when(pl.program_id(0) == 0)
    def _():
        lane = iw_ref[0] & 127
        m = lax.broadcasted_iota(jnp.int32, (D, 128), 1) == lane
        inp_ref[...] = jnp.sum(
            jnp.where(m, win_blk[...], 0.0), axis=1, keepdims=True
        )

    prod = lax.dot_general(
        inp_ref[...], wctx_blk[...], (((0,), (0,)), ((), ()))
    )
    out_ref[...] = prod.reshape(CW // 128, 128)


_tc_scan = pl.pallas_call(
    _tc_scan_body,
    grid_spec=pltpu.PrefetchScalarGridSpec(
        num_scalar_prefetch=1,
        grid=(GRID,),
        in_specs=[
            pl.BlockSpec((D, 128), lambda g, iw: (0, iw[0] // 128)),
            pl.BlockSpec((D, CW), lambda g, iw: (0, g)),
        ],
        out_specs=pl.BlockSpec((CW // 128, 128), lambda g, iw: (g, 0)),
        scratch_shapes=[pltpu.VMEM((D, 1), jnp.float32)],
    ),
    out_shape=jax.ShapeDtypeStruct((ROWS, 128), jnp.float32),
)


_mesh = plsc.VectorSubcoreMesh(core_axis_name="c", subcore_axis_name="s")

_SC_SCRATCH = [
    pltpu.VMEM((BPW,), jnp.int32),
    pltpu.VMEM((BPW, 128), jnp.float32),
    pltpu.VMEM((BPW,), jnp.float32),
    pltpu.SemaphoreType.DMA,
]


def _sc_gather_body(ctx_hbm, dots2_hbm, out_hbm, idx_v, rows_v, dots_v, sem):
    wid = lax.axis_index("s") * NC + lax.axis_index("c")
    base = wid * BPW
    pltpu.sync_copy(ctx_hbm.at[pl.ds(base, BPW)], idx_v)
    riota = lax.iota(jnp.int32, L)

    def fire(g, carry):
        rows16 = idx_v[pl.ds(g * L, L)] >> 7
        pltpu.async_copy(
            dots2_hbm.at[rows16], rows_v.at[pl.ds(g * L, L)], sem
        )
        return carry

    lax.fori_loop(0, BPW // L, fire, 0)

    def drain_extract(g, carry):
        pltpu.make_async_copy(
            dots2_hbm.at[pl.ds(0, L)], rows_v.at[pl.ds(g * L, L)], sem
        ).wait()
        lanes = idx_v[pl.ds(g * L, L)] & 127
        vals = plsc.load_gather(rows_v, [riota + g * L, lanes])
        dots_v[pl.ds(g * L, L)] = vals
        return carry

    lax.fori_loop(0, BPW // L, drain_extract, 0)

    pltpu.sync_copy(dots_v, out_hbm.at[pl.ds(base, BPW)])


_sc_gather = pl.kernel(
    _sc_gather_body,
    mesh=_mesh,
    out_type=jax.ShapeDtypeStruct((B,), jnp.float32),
    compiler_params=pltpu.CompilerParams(needs_layout_passes=False),
    scratch_types=_SC_SCRATCH,
)


def _softmax_body(x_ref, o_ref):
    x = x_ref[...]
    m = jnp.max(x)
    e = jnp.exp(x - m)
    o_ref[...] = e / jnp.sum(e)


_tc_softmax = pl.pallas_call(
    _softmax_body,
    out_shape=jax.ShapeDtypeStruct((8, B // 8), jnp.float32),
)


def kernel(input_word, context, W_in, W_ctx):
    dots2 = _tc_scan(input_word, W_in.T, W_ctx.T)
    dots = _sc_gather(context, dots2)
    scores = _tc_softmax(dots.reshape(8, B // 8))
    return scores.reshape(1, B)


# speedup vs baseline: 9.8205x; 1.3266x over previous
import jax
import jax.numpy as jnp
from jax import lax
from jax.experimental import pallas as pl
from jax.experimental.pallas import tpu as pltpu
from jax.experimental.pallas import tpu_sc as plsc

VOCAB = 1000000
D = 64
B = 16384
NC = 2
NS = 16
NW = NC * NS
BPW = B // NW
L = 16

CW = 16384
GRID = -(-VOCAB // CW)
ROWS = GRID * (CW // 128)


def _tc_scan_body(iw_ref, win_blk, wctx_blk, out_ref, inp_ref):
    @pl.when(pl.program_id(0) == 0)
    def _():
        lane = iw_ref[0] & 127
        m = lax.broadcasted_iota(jnp.int32, (D, 128), 1) == lane
        inp_ref[...] = jnp.sum(
            jnp.where(m, win_blk[...], 0.0), axis=1, keepdims=True
        )

    prod = lax.dot_general(
        inp_ref[...], wctx_blk[...], (((0,), (0,)), ((), ()))
    )
    out_ref[...] = prod.reshape(CW // 128, 128)


_tc_scan = pl.pallas_call(
    _tc_scan_body,
    grid_spec=pltpu.PrefetchScalarGridSpec(
        num_scalar_prefetch=1,
        grid=(GRID,),
        in_specs=[
            pl.BlockSpec((D, 128), lambda g, iw: (0, iw[0] // 128)),
            pl.BlockSpec((D, CW), lambda g, iw: (0, g)),
        ],
        out_specs=pl.BlockSpec((CW // 128, 128), lambda g, iw: (g, 0)),
        scratch_shapes=[pltpu.VMEM((D, 1), jnp.float32)],
    ),
    out_shape=jax.ShapeDtypeStruct((ROWS, 128), jnp.float32),
)


_mesh = plsc.VectorSubcoreMesh(core_axis_name="c", subcore_axis_name="s")

_SC_SCRATCH = [
    pltpu.VMEM((BPW,), jnp.int32),
    pltpu.VMEM((BPW, 128), jnp.float32),
    pltpu.VMEM((BPW,), jnp.float32),
    pltpu.SemaphoreType.DMA,
]


def _sc_gather_body(ctx_hbm, dots2_hbm, out_hbm, idx_v, rows_v, dots_v, sem):
    wid = lax.axis_index("s") * NC + lax.axis_index("c")
    base = wid * BPW
    pltpu.sync_copy(ctx_hbm.at[pl.ds(base, BPW)], idx_v)
    riota = lax.iota(jnp.int32, L)

    def fire(g, carry):
        rows16 = idx_v[pl.ds(g * L, L)] >> 7
        pltpu.async_copy(
            dots2_hbm.at[rows16], rows_v.at[pl.ds(g * L, L)], sem
        )
        return carry

    lax.fori_loop(0, BPW // L, fire, 0)

    def drain_extract(g, carry):
        pltpu.make_async_copy(
            dots2_hbm.at[pl.ds(0, L)], rows_v.at[pl.ds(g * L, L)], sem
        ).wait()
        lanes = idx_v[pl.ds(g * L, L)] & 127
        vals = plsc.load_gather(rows_v, [riota + g * L, lanes])
        dots_v[pl.ds(g * L, L)] = vals
        return carry

    lax.fori_loop(0, BPW // L, drain_extract, 0)

    pltpu.sync_copy(dots_v, out_hbm.at[pl.ds(base, BPW)])


_sc_gather = pl.kernel(
    _sc_gather_body,
    mesh=_mesh,
    out_type=jax.ShapeDtypeStruct((B,), jnp.float32),
    compiler_params=pltpu.CompilerParams(needs_layout_passes=False),
    scratch_types=_SC_SCRATCH,
)


def _softmax_body(x_ref, o_ref):
    x = x_ref[...]
    m = jnp.max(x)
    e = jnp.exp(x - m)
    o_ref[...] = e / jnp.sum(e)


_tc_softmax = pl.pallas_call(
    _softmax_body,
    out_shape=jax.ShapeDtypeStruct((8, B // 8), jnp.float32),
)


def kernel(input_word, context, W_in, W_ctx):
    dots2 = _tc_scan(input_word, W_in.T, W_ctx.T)
    dots = _sc_gather(context, dots2)
    scores = _tc_softmax(dots.reshape(8, B // 8))
    return scores.reshape(1, B)


# speedup vs baseline: 10.9328x; 1.1133x over previous
import jax
import jax.numpy as jnp
from jax import lax
from jax.experimental import pallas as pl
from jax.experimental.pallas import tpu as pltpu
from jax.experimental.pallas import tpu_sc as plsc

VOCAB = 1000000
D = 64
B = 16384
NC = 2
NS = 16
NW = NC * NS
BPW = B // NW
L = 16

CW = 32768
GRID = -(-VOCAB // CW)
ROWS = GRID * (CW // 128)


def _tc_scan_body(iw_ref, win_blk, wctx_blk, out_ref, inp_ref):
    @pl.when(pl.program_id(0) == 0)
    def _():
        lane = iw_ref[0] & 127
        m = lax.broadcasted_iota(jnp.int32, (D, 128), 1) == lane
        inp_ref[...] = jnp.sum(
            jnp.where(m, win_blk[...], 0.0), axis=1, keepdims=True
        )

    prod = lax.dot_general(
        inp_ref[...], wctx_blk[...], (((0,), (0,)), ((), ()))
    )
    out_ref[...] = prod.reshape(CW // 128, 128)


_tc_scan = pl.pallas_call(
    _tc_scan_body,
    grid_spec=pltpu.PrefetchScalarGridSpec(
        num_scalar_prefetch=1,
        grid=(GRID,),
        in_specs=[
            pl.BlockSpec((D, 128), lambda g, iw: (0, iw[0] // 128)),
            pl.BlockSpec((D, CW), lambda g, iw: (0, g)),
        ],
        out_specs=pl.BlockSpec((CW // 128, 128), lambda g, iw: (g, 0)),
        scratch_shapes=[pltpu.VMEM((D, 1), jnp.float32)],
    ),
    out_shape=jax.ShapeDtypeStruct((ROWS, 128), jnp.float32),
)


_mesh = plsc.VectorSubcoreMesh(core_axis_name="c", subcore_axis_name="s")

_SC_SCRATCH = [
    pltpu.VMEM((BPW,), jnp.int32),
    pltpu.VMEM((BPW, 128), jnp.float32),
    pltpu.VMEM((BPW,), jnp.float32),
    pltpu.SemaphoreType.DMA,
]


def _sc_gather_body(ctx_hbm, dots2_hbm, out_hbm, idx_v, rows_v, dots_v, sem):
    wid = lax.axis_index("s") * NC + lax.axis_index("c")
    base = wid * BPW
    pltpu.sync_copy(ctx_hbm.at[pl.ds(base, BPW)], idx_v)
    riota = lax.iota(jnp.int32, L)

    def fire(g, carry):
        rows16 = idx_v[pl.ds(g * L, L)] >> 7
        pltpu.async_copy(
            dots2_hbm.at[rows16], rows_v.at[pl.ds(g * L, L)], sem
        )
        return carry

    lax.fori_loop(0, BPW // L, fire, 0)

    def drain_extract(g, carry):
        pltpu.make_async_copy(
            dots2_hbm.at[pl.ds(0, L)], rows_v.at[pl.ds(g * L, L)], sem
        ).wait()
        lanes = idx_v[pl.ds(g * L, L)] & 127
        vals = plsc.load_gather(rows_v, [riota + g * L, lanes])
        dots_v[pl.ds(g * L, L)] = vals
        return carry

    lax.fori_loop(0, BPW // L, drain_extract, 0)

    pltpu.sync_copy(dots_v, out_hbm.at[pl.ds(base, BPW)])


_sc_gather = pl.kernel(
    _sc_gather_body,
    mesh=_mesh,
    out_type=jax.ShapeDtypeStruct((B,), jnp.float32),
    compiler_params=pltpu.CompilerParams(needs_layout_passes=False),
    scratch_types=_SC_SCRATCH,
)


def _softmax_body(x_ref, o_ref):
    x = x_ref[...]
    m = jnp.max(x)
    e = jnp.exp(x - m)
    o_ref[...] = e / jnp.sum(e)


_tc_softmax = pl.pallas_call(
    _softmax_body,
    out_shape=jax.ShapeDtypeStruct((8, B // 8), jnp.float32),
)


def kernel(input_word, context, W_in, W_ctx):
    dots2 = _tc_scan(input_word, W_in.T, W_ctx.T)
    dots = _sc_gather(context, dots2)
    scores = _tc_softmax(dots.reshape(8, B // 8))
    return scores.reshape(1, B)
